# baseline (speedup 1.0000x reference)
import functools

import jax
import jax.numpy as jnp
from jax import lax
from jax.experimental import pallas as pl
from jax.experimental.pallas import tpu as pltpu
from jax.experimental.pallas import tpu_sc as plsc

EMBED_DIM = 64
HIDDEN_DIM = 256
VOCAB_SIZE = 1000000
CATEGORY_SIZE = 1000
CAT_PAD = 1024
BATCH = 16384

NUM_CORES = 2
NUM_SUBCORES = 16
NUM_WORKERS = NUM_CORES * NUM_SUBCORES
B_PER_W = BATCH // NUM_WORKERS
LANES = 16
RING = 16
ROUNDS = B_PER_W // RING


def _sc_title_body(idx_hbm, tab_hbm, out_hbm, idx_v, eidx_v, rows_v, sem):
    wid = lax.axis_index("s") * NUM_CORES + lax.axis_index("c")
    base = wid * B_PER_W
    pltpu.sync_copy(idx_hbm.at[pl.ds(base, B_PER_W)], idx_v)
    iota = lax.iota(jnp.int32, LANES)

    def round_body(r, carry):
        copies = []
        for s in range(RING):
            e = r * RING + s
            v_splat = plsc.load_gather(idx_v, [jnp.full((LANES,), 0, jnp.int32) + e])
            for c in range(EMBED_DIM // LANES):
                eidx_v[pl.ds(s * EMBED_DIM + c * LANES, LANES)] = (
                    (iota + c * LANES) * VOCAB_SIZE + v_splat)
            copies.append(pltpu.async_copy(
                tab_hbm.at[eidx_v.at[pl.ds(s * EMBED_DIM, EMBED_DIM)]],
                rows_v.at[pl.ds(e * EMBED_DIM, EMBED_DIM)], sem))
        for cp in copies:
            cp.wait()
        return carry

    lax.fori_loop(0, ROUNDS, round_body, 0)
    pltpu.sync_copy(rows_v, out_hbm.at[pl.ds(base * EMBED_DIM, B_PER_W * EMBED_DIM)])


_sc_title_gather = functools.partial(
    pl.kernel,
    mesh=plsc.VectorSubcoreMesh(
        core_axis_name="c", subcore_axis_name="s",
        num_cores=NUM_CORES, num_subcores=NUM_SUBCORES),
    out_type=jax.ShapeDtypeStruct((BATCH * EMBED_DIM,), jnp.float32),
    scratch_types=[
        pltpu.VMEM((B_PER_W,), jnp.int32),
        pltpu.VMEM((RING * EMBED_DIM,), jnp.int32),
        pltpu.VMEM((B_PER_W * EMBED_DIM,), jnp.float32),
        pltpu.SemaphoreType.DMA,
    ],
    compiler_params=pltpu.CompilerParams(needs_layout_passes=False),
)(_sc_title_body)


def _mlp_body(x1_ref, cidx_ref, ctab_ref, w1a_ref, w1b_ref, b1_ref, w2_ref,
              b2_ref, o_ref):
    m = x1_ref.shape[0]
    onehot = (jax.lax.broadcasted_iota(jnp.int32, (m, CAT_PAD), 1)
              == cidx_ref[...]).astype(jnp.float32)
    x2 = jnp.dot(onehot, ctab_ref[...], preferred_element_type=jnp.float32)
    h = jnp.dot(x1_ref[...], w1a_ref[...], preferred_element_type=jnp.float32)
    h = h + jnp.dot(x2, w1b_ref[...], preferred_element_type=jnp.float32)
    h = jnp.maximum(h + b1_ref[...], 0.0)
    out = jnp.dot(h, w2_ref[...], preferred_element_type=jnp.float32) + b2_ref[...]
    norm = jnp.sqrt(jnp.sum(out * out, axis=1, keepdims=True))
    o_ref[...] = out / jnp.maximum(norm, 1e-12)


def _mlp(trows, cidx, ctab_pad, w1a, w1b, b1, w2, b2, block_m=2048):
    grid = (BATCH // block_m,)
    return pl.pallas_call(
        _mlp_body,
        grid=grid,
        in_specs=[
            pl.BlockSpec((block_m, EMBED_DIM), lambda i: (i, 0)),
            pl.BlockSpec((block_m, 1), lambda i: (i, 0)),
            pl.BlockSpec((CAT_PAD, EMBED_DIM), lambda i: (0, 0)),
            pl.BlockSpec((EMBED_DIM, HIDDEN_DIM), lambda i: (0, 0)),
            pl.BlockSpec((EMBED_DIM, HIDDEN_DIM), lambda i: (0, 0)),
            pl.BlockSpec((1, HIDDEN_DIM), lambda i: (0, 0)),
            pl.BlockSpec((HIDDEN_DIM, EMBED_DIM), lambda i: (0, 0)),
            pl.BlockSpec((1, EMBED_DIM), lambda i: (0, 0)),
        ],
        out_specs=pl.BlockSpec((block_m, EMBED_DIM), lambda i: (i, 0)),
        out_shape=jax.ShapeDtypeStruct((BATCH, EMBED_DIM), jnp.float32),
    )(trows, cidx, ctab_pad, w1a, w1b, b1, w2, b2)


def kernel(title_idx, category_idx, title_table, category_table, W1, b1, W2, b2):
    tidx = title_idx.astype(jnp.int32)
    tab_flat = title_table.T.reshape(-1)
    trows_flat = _sc_title_gather(tidx, tab_flat)
    trows = trows_flat.reshape(BATCH, EMBED_DIM)
    ctab_pad = jnp.zeros((CAT_PAD, EMBED_DIM), jnp.float32).at[:CATEGORY_SIZE].set(
        category_table)
    w1t = W1.T
    return _mlp(trows, category_idx.astype(jnp.int32).reshape(BATCH, 1),
                ctab_pad, w1t[:EMBED_DIM], w1t[EMBED_DIM:],
                b1.reshape(1, HIDDEN_DIM), W2.T, b2.reshape(1, EMBED_DIM))
